# baseline (speedup 1.0000x reference)
import functools

import jax
import jax.numpy as jnp
from jax import lax
from jax.experimental import pallas as pl
from jax.experimental.pallas import tpu as pltpu
from jax.experimental.pallas import tpu_sc as plsc

_NG, _P, _NPC, _FDIM = 10, 7, 23, 70
_N = _P * _NPC
_NN = _N * _N
_TOT = _NG * _NN
_NROWS = _NG * _N
_GROUPS = 201
_GW = 8 * _N
_GBUF = _GW + 24
_GPT = 7
_LEFT_ROW = 8 * _GROUPS
_LEFT_OFF = _LEFT_ROW * _N
_TAIL_START = _TOT - 16
_NW = 32
_ROWPAT = [tuple((16 * v + l) // _NPC for l in range(16)) for v in range(10)]

_mesh = plsc.VectorSubcoreMesh(core_axis_name="c", subcore_axis_name="s")


@functools.partial(
    pl.kernel,
    mesh=_mesh,
    compiler_params=pltpu.CompilerParams(needs_layout_passes=False),
    out_type=jax.ShapeDtypeStruct((_TOT,), jnp.float32),
    scratch_types=[
        pltpu.VMEM((_NG * 49,), jnp.float32),
        pltpu.VMEM((_NG * 49,), jnp.float32),
        pltpu.VMEM((_NG * 49,), jnp.float32),
        pltpu.VMEM((384,), jnp.float32),
        pltpu.VMEM((_GPT * _GBUF,), jnp.float32),
        pltpu.VMEM((128,), jnp.float32),
        pltpu.VMEM((128,), jnp.float32),
        pltpu.VMEM((16,), jnp.int32),
        pltpu.VMEM((16,), jnp.float32),
        pltpu.SemaphoreType.DMA,
        pltpu.SemaphoreType.DMA,
    ],
)
def _sc_rescale(hc_hbm, rc_hbm, rcs_hbm, w_hbm, wout_hbm, hc_v, rc_v, rcs_v,
                rp_v, wg_v, tmp_v, tmp2_v, tidx_v, tval_v, sem_in, sem_out):
    nc = 2
    wid = lax.axis_index("s") * nc + lax.axis_index("c")
    is_last = wid == _NW - 1
    sg = jnp.minimum(7 * wid, 6 * wid + 9)

    lanes = lax.broadcasted_iota(jnp.int32, (16,), 0)
    minl = jnp.minimum(lanes, 6)

    in_copies = []
    for gi in range(_GPT):
        gb = jnp.minimum(sg + gi, _GROUPS - 1)
        in_copies.append(pltpu.async_copy(
            w_hbm.at[pl.ds(gb * _GW, _GW)],
            wg_v.at[pl.ds(gi * _GBUF, _GW)], sem_in))

    pltpu.sync_copy(hc_hbm, hc_v)
    pltpu.sync_copy(rc_hbm, rc_v)
    pltpu.sync_copy(rcs_hbm, rcs_v)

    ga = jnp.clip((sg * 8) // _N, 0, _NG - 2)

    for m in range(2):
        base = (ga + m) * 49
        s = jnp.zeros((16,), jnp.float32)
        for r in range(7):
            s = s + plsc.load_gather(hc_v, [base + r * 7 + minl])
        tmp_v[pl.ds(16, 16)] = s
        dh = jnp.zeros((16,), jnp.float32)
        for k in range(7):
            sk = plsc.load_gather(tmp_v, [jnp.full((16,), 16 + k, jnp.int32)])
            dh = dh + sk * plsc.load_gather(rc_v, [base + k * 7 + minl])
        dh = 0.1 * dh
        rdiag = plsc.load_gather(rcs_v, [base + minl * 8])
        t = dh / rdiag
        tmp2_v[pl.ds(16, 16)] = t
        for a in range(7):
            rc_row = plsc.load_gather(rc_v, [base + a * 7 + minl])
            rc_col = plsc.load_gather(rc_v, [base + minl * 7 + a])
            ta = plsc.load_gather(tmp2_v, [jnp.full((16,), 16 + a, jnp.int32)])
            row = 1.0 + t + (rc_col / rc_row) * ta
            row = jnp.where(lanes == a, 1.0, row)
            rp_v[pl.ds(16 + m * 128 + a * 16, 16)] = row

    def do_row(wofs, r):
        g = r // _N
        i = r - g * _N
        rbase = 16 + jnp.clip(g - ga, 0, 1) * 128 + (i // _NPC) * 16
        for v in range(10):
            cv = (16 * v) // _NPC
            cut = _NPC * (cv + 1) - 16 * v
            if cut >= 16:
                idxv = (rbase + cv) + lanes * 0
            else:
                idxv = rbase + jnp.where(lanes < cut, cv, cv + 1)
            w = wg_v[pl.ds(wofs + 16 * v, 16)]
            sc = plsc.load_gather(rp_v, [idxv])
            wg_v[pl.ds(wofs + 16 * v, 16)] = w * sc
        w = wg_v[pl.ds(wofs + 160, 16)]
        sc = plsc.load_gather(rp_v, [(rbase + 6) + lanes * 0])
        plsc.store_scatter(wg_v, [wofs + 160 + lanes], w * sc, mask=lanes < 1)

    out_copies = []
    for gi in range(_GPT):
        gb = jnp.minimum(sg + gi, _GROUPS - 1)
        in_copies[gi].wait()
        for k in range(8):
            do_row(gi * _GBUF + k * _N, gb * 8 + k)
        out_copies.append(pltpu.async_copy(
            wg_v.at[pl.ds(gi * _GBUF, _GW)],
            wout_hbm.at[pl.ds(gb * _GW, _GW)], sem_out))
    for c in out_copies:
        c.wait()

    @pl.when(is_last)
    def _():
        pltpu.sync_copy(w_hbm.at[pl.ds(_LEFT_OFF, 320)],
                        wg_v.at[pl.ds(0, 320)])
        do_row(0, _LEFT_ROW)
        do_row(_N, _LEFT_ROW + 1)
        pltpu.sync_copy(wg_v.at[pl.ds(0, 320)],
                        wout_hbm.at[pl.ds(_LEFT_OFF, 320)])
        tidx_v[...] = _TAIL_START + lanes
        pltpu.async_copy(w_hbm.at[tidx_v], tval_v, sem_in).wait()
        f = _TAIL_START + lanes
        g = f // _NN
        r = f - g * _NN
        i = r // _N
        j = r - i * _N
        idx = 16 + jnp.clip(g - ga, 0, 1) * 128 + (i // _NPC) * 16 + (j // _NPC)
        sc = plsc.load_gather(rp_v, [idx])
        tval_v[...] = tval_v[...] * sc
        pltpu.async_copy(tval_v, wout_hbm.at[tidx_v], sem_out).wait()


def _stage2_kernel(wp_ref, rn_ref, hp_ref, d_ref, theta_ref, hpk_ref):
    d_diag = jnp.sum(d_ref[...] * jnp.eye(_NPC, dtype=jnp.float32), axis=-1)
    r = lax.rsqrt(d_diag)
    a = (wp_ref[...] * rn_ref[...]
         * r[:, :, :, None] * r[:, :, None, :])
    a = a.reshape(_NG * _P, _NPC, _NPC)
    hp = hp_ref[...].reshape(_NG * _P, _NPC, _FDIM)
    ahp = lax.dot_general(a, hp,
                          (((2,), (1,)), ((0,), (0,))),
                          preferred_element_type=jnp.float32)
    hpk = lax.dot_general(ahp, theta_ref[...],
                          (((2,), (0,)), ((), ())),
                          preferred_element_type=jnp.float32)
    hpk_ref[...] = 0.1 * hpk.reshape(_NG, _P, _NPC, _FDIM)


def _stage2(Wp, Rn, Hp, D, theta):
    return pl.pallas_call(
        _stage2_kernel,
        out_shape=jax.ShapeDtypeStruct((_NG, _P, _NPC, _FDIM), jnp.float32),
    )(Wp, Rn, Hp, D, theta)


def kernel(Hc, Rc, Rcs, Wp, Rn, Hp, D, W, theta):
    w_out_flat = _sc_rescale(Hc.reshape(-1), Rc.reshape(-1), Rcs.reshape(-1),
                             W.reshape(-1))
    hp_k = _stage2(Wp, Rn, Hp, D, theta)
    return (w_out_flat.reshape(_NG, _N, _N), hp_k)


# speedup vs baseline: 1.2454x; 1.2454x over previous
import functools

import jax
import jax.numpy as jnp
from jax import lax
from jax.experimental import pallas as pl
from jax.experimental.pallas import tpu as pltpu
from jax.experimental.pallas import tpu_sc as plsc

_NG, _P, _NPC, _FDIM = 10, 7, 23, 70
_N = _P * _NPC
_NN = _N * _N
_TOT = _NG * _NN
_NROWS = _NG * _N
_GROUPS = 201
_GW = 8 * _N
_GBUF = _GW + 24
_GPT = 7
_LEFT_ROW = 8 * _GROUPS
_LEFT_OFF = _LEFT_ROW * _N
_TAIL_START = _TOT - 16
_NW = 32
_ROWPAT = [tuple((16 * v + l) // _NPC for l in range(16)) for v in range(10)]

_mesh = plsc.VectorSubcoreMesh(core_axis_name="c", subcore_axis_name="s")


@functools.partial(
    pl.kernel,
    mesh=_mesh,
    compiler_params=pltpu.CompilerParams(needs_layout_passes=False),
    out_type=jax.ShapeDtypeStruct((_TOT,), jnp.float32),
    scratch_types=[
        pltpu.VMEM((_NG * 49,), jnp.float32),
        pltpu.VMEM((_NG * 49,), jnp.float32),
        pltpu.VMEM((_NG * 49,), jnp.float32),
        pltpu.VMEM((384,), jnp.float32),
        pltpu.VMEM((_GPT * _GBUF,), jnp.float32),
        pltpu.VMEM((128,), jnp.float32),
        pltpu.VMEM((128,), jnp.float32),
        pltpu.VMEM((16,), jnp.int32),
        pltpu.VMEM((16,), jnp.float32),
        pltpu.SemaphoreType.DMA,
        pltpu.SemaphoreType.DMA,
    ],
)
def _sc_rescale(hc_hbm, rc_hbm, rcs_hbm, w_hbm, wout_hbm, hc_v, rc_v, rcs_v,
                rp_v, wg_v, tmp_v, tmp2_v, tidx_v, tval_v, sem_in, sem_out):
    nc = 2
    wid = lax.axis_index("s") * nc + lax.axis_index("c")
    is_last = wid == _NW - 1
    sg = jnp.minimum(7 * wid, 6 * wid + 9)

    lanes = lax.broadcasted_iota(jnp.int32, (16,), 0)
    minl = jnp.minimum(lanes, 6)

    in_copies = []
    for gi in range(_GPT):
        gb = jnp.minimum(sg + gi, _GROUPS - 1)
        in_copies.append(pltpu.async_copy(
            w_hbm.at[pl.ds(gb * _GW, _GW)],
            wg_v.at[pl.ds(gi * _GBUF, _GW)], sem_in))

    s1_copies = [pltpu.async_copy(hc_hbm, hc_v, sem_out),
                 pltpu.async_copy(rc_hbm, rc_v, sem_out),
                 pltpu.async_copy(rcs_hbm, rcs_v, sem_out)]
    for c in s1_copies:
        c.wait()

    ga = jnp.clip((sg * 8) // _N, 0, _NG - 2)

    for m in range(2):
        base = (ga + m) * 49
        s = jnp.zeros((16,), jnp.float32)
        for r in range(7):
            s = s + plsc.load_gather(hc_v, [base + r * 7 + minl])
        tmp_v[pl.ds(16, 16)] = s
        dh = jnp.zeros((16,), jnp.float32)
        for k in range(7):
            sk = plsc.load_gather(tmp_v, [jnp.full((16,), 16 + k, jnp.int32)])
            dh = dh + sk * plsc.load_gather(rc_v, [base + k * 7 + minl])
        dh = 0.1 * dh
        rdiag = plsc.load_gather(rcs_v, [base + minl * 8])
        t = dh / rdiag
        tmp2_v[pl.ds(16, 16)] = t
        for a in range(7):
            rc_row = plsc.load_gather(rc_v, [base + a * 7 + minl])
            rc_col = plsc.load_gather(rc_v, [base + minl * 7 + a])
            ta = plsc.load_gather(tmp2_v, [jnp.full((16,), 16 + a, jnp.int32)])
            row = 1.0 + t + (rc_col / rc_row) * ta
            row = jnp.where(lanes == a, 1.0, row)
            rp_v[pl.ds(16 + m * 128 + a * 16, 16)] = row

    def scale_vecs(rbase):
        out = []
        for v in range(10):
            cv = (16 * v) // _NPC
            cut = _NPC * (cv + 1) - 16 * v
            if cut >= 16:
                idxv = (rbase + cv) + lanes * 0
            else:
                idxv = rbase + jnp.where(lanes < cut, cv, cv + 1)
            out.append(plsc.load_gather(rp_v, [idxv]))
        out.append(plsc.load_gather(rp_v, [(rbase + 6) + lanes * 0]))
        return out

    def do_row(wofs, r):
        g = r // _N
        i = r - g * _N
        sv = scale_vecs(16 + jnp.clip(g - ga, 0, 1) * 128 + (i // _NPC) * 16)
        for v in range(10):
            w = wg_v[pl.ds(wofs + 16 * v, 16)]
            wg_v[pl.ds(wofs + 16 * v, 16)] = w * sv[v]
        w = wg_v[pl.ds(wofs + 160, 16)]
        plsc.store_scatter(wg_v, [wofs + 160 + lanes], w * sv[10],
                           mask=lanes < 1)

    out_copies = []
    for gi in range(_GPT):
        gb = jnp.minimum(sg + gi, _GROUPS - 1)
        r0 = gb * 8
        g_a = r0 // _N
        i_a = r0 - g_a * _N
        pi_a = i_a // _NPC
        rb_a = 16 + jnp.clip(g_a - ga, 0, 1) * 128 + pi_a * 16
        r7 = r0 + 7
        g_b = r7 // _N
        i_b = r7 - g_b * _N
        rb_b = 16 + jnp.clip(g_b - ga, 0, 1) * 128 + (i_b // _NPC) * 16
        kb = _NPC * (pi_a + 1) - i_a
        in_copies[gi].wait()
        sv_a = scale_vecs(rb_a)
        sv_b = scale_vecs(rb_b)
        for k in range(8):
            use_a = k < kb
            wofs = gi * _GBUF + k * _N
            for v in range(10):
                w = wg_v[pl.ds(wofs + 16 * v, 16)]
                wg_v[pl.ds(wofs + 16 * v, 16)] = (
                    w * jnp.where(use_a, sv_a[v], sv_b[v]))
            w = wg_v[pl.ds(wofs + 160, 16)]
            plsc.store_scatter(wg_v, [wofs + 160 + lanes],
                               w * jnp.where(use_a, sv_a[10], sv_b[10]),
                               mask=lanes < 1)
        out_copies.append(pltpu.async_copy(
            wg_v.at[pl.ds(gi * _GBUF, _GW)],
            wout_hbm.at[pl.ds(gb * _GW, _GW)], sem_out))
    for c in out_copies:
        c.wait()

    @pl.when(is_last)
    def _():
        pltpu.sync_copy(w_hbm.at[pl.ds(_LEFT_OFF, 320)],
                        wg_v.at[pl.ds(0, 320)])
        do_row(0, _LEFT_ROW)
        do_row(_N, _LEFT_ROW + 1)
        pltpu.sync_copy(wg_v.at[pl.ds(0, 320)],
                        wout_hbm.at[pl.ds(_LEFT_OFF, 320)])
        tidx_v[...] = _TAIL_START + lanes
        pltpu.async_copy(w_hbm.at[tidx_v], tval_v, sem_in).wait()
        f = _TAIL_START + lanes
        g = f // _NN
        r = f - g * _NN
        i = r // _N
        j = r - i * _N
        idx = 16 + jnp.clip(g - ga, 0, 1) * 128 + (i // _NPC) * 16 + (j // _NPC)
        sc = plsc.load_gather(rp_v, [idx])
        tval_v[...] = tval_v[...] * sc
        pltpu.async_copy(tval_v, wout_hbm.at[tidx_v], sem_out).wait()


def _stage2_kernel(wp_ref, rn_ref, hp_ref, d_ref, theta_ref, hpk_ref):
    d_diag = jnp.sum(d_ref[...] * jnp.eye(_NPC, dtype=jnp.float32), axis=-1)
    r = lax.rsqrt(d_diag)
    a = (wp_ref[...] * rn_ref[...]
         * r[:, :, :, None] * r[:, :, None, :])
    a = a.reshape(_NG * _P, _NPC, _NPC)
    hp = hp_ref[...].reshape(_NG * _P, _NPC, _FDIM)
    ahp = lax.dot_general(a, hp,
                          (((2,), (1,)), ((0,), (0,))),
                          preferred_element_type=jnp.float32)
    hpk = lax.dot_general(ahp, theta_ref[...],
                          (((2,), (0,)), ((), ())),
                          preferred_element_type=jnp.float32)
    hpk_ref[...] = 0.1 * hpk.reshape(_NG, _P, _NPC, _FDIM)


def _stage2(Wp, Rn, Hp, D, theta):
    return pl.pallas_call(
        _stage2_kernel,
        out_shape=jax.ShapeDtypeStruct((_NG, _P, _NPC, _FDIM), jnp.float32),
    )(Wp, Rn, Hp, D, theta)


def kernel(Hc, Rc, Rcs, Wp, Rn, Hp, D, W, theta):
    w_out_flat = _sc_rescale(Hc.reshape(-1), Rc.reshape(-1), Rcs.reshape(-1),
                             W.reshape(-1))
    hp_k = _stage2(Wp, Rn, Hp, D, theta)
    return (w_out_flat.reshape(_NG, _N, _N), hp_k)


# speedup vs baseline: 1.2668x; 1.0172x over previous
import functools

import jax
import jax.numpy as jnp
from jax import lax
from jax.experimental import pallas as pl
from jax.experimental.pallas import tpu as pltpu
from jax.experimental.pallas import tpu_sc as plsc

_NG, _P, _NPC, _FDIM = 10, 7, 23, 70
_N = _P * _NPC
_NN = _N * _N
_TOT = _NG * _NN
_NROWS = _NG * _N
_GROUPS = 201
_GW = 8 * _N
_GBUF = _GW + 24
_GPT = 7
_LEFT_ROW = 8 * _GROUPS
_LEFT_OFF = _LEFT_ROW * _N
_TAIL_START = _TOT - 16
_NW = 32
_ROWPAT = [tuple((16 * v + l) // _NPC for l in range(16)) for v in range(10)]

_mesh = plsc.VectorSubcoreMesh(core_axis_name="c", subcore_axis_name="s")


@functools.partial(
    pl.kernel,
    mesh=_mesh,
    compiler_params=pltpu.CompilerParams(needs_layout_passes=False),
    out_type=jax.ShapeDtypeStruct((_TOT,), jnp.float32),
    scratch_types=[
        pltpu.VMEM((3 * _NG * 49,), jnp.float32),
        pltpu.VMEM((384,), jnp.float32),
        pltpu.VMEM((_GPT * _GBUF,), jnp.float32),
        pltpu.VMEM((128,), jnp.float32),
        pltpu.VMEM((128,), jnp.float32),
        pltpu.VMEM((16,), jnp.int32),
        pltpu.VMEM((16,), jnp.float32),
        pltpu.SemaphoreType.DMA,
        pltpu.SemaphoreType.DMA,
    ],
)
def _sc_rescale(s1_hbm, w_hbm, wout_hbm, s1_v,
                rp_v, wg_v, tmp_v, tmp2_v, tidx_v, tval_v, sem_in, sem_out):
    nc = 2
    wid = lax.axis_index("s") * nc + lax.axis_index("c")
    is_last = wid == _NW - 1
    sg = jnp.minimum(7 * wid, 6 * wid + 9)

    lanes = lax.broadcasted_iota(jnp.int32, (16,), 0)
    minl = jnp.minimum(lanes, 6)

    in_copies = []
    for gi in range(_GPT):
        gb = jnp.minimum(sg + gi, _GROUPS - 1)
        in_copies.append(pltpu.async_copy(
            w_hbm.at[pl.ds(gb * _GW, _GW)],
            wg_v.at[pl.ds(gi * _GBUF, _GW)], sem_in))

    pltpu.sync_copy(s1_hbm, s1_v)

    ga = jnp.clip((sg * 8) // _N, 0, _NG - 2)

    for m in range(2):
        base = (ga + m) * 49
        s = jnp.zeros((16,), jnp.float32)
        for r in range(7):
            s = s + plsc.load_gather(s1_v, [base + r * 7 + minl])
        tmp_v[pl.ds(16, 16)] = s
        dh = jnp.zeros((16,), jnp.float32)
        for k in range(7):
            sk = plsc.load_gather(tmp_v, [jnp.full((16,), 16 + k, jnp.int32)])
            dh = dh + sk * plsc.load_gather(s1_v, [490 + base + k * 7 + minl])
        dh = 0.1 * dh
        rdiag = plsc.load_gather(s1_v, [980 + base + minl * 8])
        t = dh / rdiag
        tmp2_v[pl.ds(16, 16)] = t
        for a in range(7):
            rc_row = plsc.load_gather(s1_v, [490 + base + a * 7 + minl])
            rc_col = plsc.load_gather(s1_v, [490 + base + minl * 7 + a])
            ta = plsc.load_gather(tmp2_v, [jnp.full((16,), 16 + a, jnp.int32)])
            row = 1.0 + t + (rc_col / rc_row) * ta
            row = jnp.where(lanes == a, 1.0, row)
            rp_v[pl.ds(16 + m * 128 + a * 16, 16)] = row

    def scale_vecs(rbase):
        out = []
        for v in range(10):
            cv = (16 * v) // _NPC
            cut = _NPC * (cv + 1) - 16 * v
            if cut >= 16:
                idxv = (rbase + cv) + lanes * 0
            else:
                idxv = rbase + jnp.where(lanes < cut, cv, cv + 1)
            out.append(plsc.load_gather(rp_v, [idxv]))
        out.append(plsc.load_gather(rp_v, [(rbase + 6) + lanes * 0]))
        return out

    def do_row(wofs, r):
        g = r // _N
        i = r - g * _N
        sv = scale_vecs(16 + jnp.clip(g - ga, 0, 1) * 128 + (i // _NPC) * 16)
        for v in range(10):
            w = wg_v[pl.ds(wofs + 16 * v, 16)]
            wg_v[pl.ds(wofs + 16 * v, 16)] = w * sv[v]
        w = wg_v[pl.ds(wofs + 160, 16)]
        plsc.store_scatter(wg_v, [wofs + 160 + lanes], w * sv[10],
                           mask=lanes < 1)

    out_copies = []
    for gi in range(_GPT):
        gb = jnp.minimum(sg + gi, _GROUPS - 1)
        r0 = gb * 8
        g_a = r0 // _N
        i_a = r0 - g_a * _N
        pi_a = i_a // _NPC
        rb_a = 16 + jnp.clip(g_a - ga, 0, 1) * 128 + pi_a * 16
        r7 = r0 + 7
        g_b = r7 // _N
        i_b = r7 - g_b * _N
        rb_b = 16 + jnp.clip(g_b - ga, 0, 1) * 128 + (i_b // _NPC) * 16
        kb = _NPC * (pi_a + 1) - i_a
        in_copies[gi].wait()
        sv_a = scale_vecs(rb_a)
        sv_b = scale_vecs(rb_b)
        for k in range(8):
            use_a = k < kb
            wofs = gi * _GBUF + k * _N
            for v in range(10):
                w = wg_v[pl.ds(wofs + 16 * v, 16)]
                wg_v[pl.ds(wofs + 16 * v, 16)] = (
                    w * jnp.where(use_a, sv_a[v], sv_b[v]))
            w = wg_v[pl.ds(wofs + 160, 16)]
            plsc.store_scatter(wg_v, [wofs + 160 + lanes],
                               w * jnp.where(use_a, sv_a[10], sv_b[10]),
                               mask=lanes < 1)
        out_copies.append(pltpu.async_copy(
            wg_v.at[pl.ds(gi * _GBUF, _GW)],
            wout_hbm.at[pl.ds(gb * _GW, _GW)], sem_out))
    for c in out_copies:
        c.wait()

    @pl.when(is_last)
    def _():
        pltpu.sync_copy(w_hbm.at[pl.ds(_LEFT_OFF, 320)],
                        wg_v.at[pl.ds(0, 320)])
        do_row(0, _LEFT_ROW)
        do_row(_N, _LEFT_ROW + 1)
        pltpu.sync_copy(wg_v.at[pl.ds(0, 320)],
                        wout_hbm.at[pl.ds(_LEFT_OFF, 320)])
        tidx_v[...] = _TAIL_START + lanes
        pltpu.async_copy(w_hbm.at[tidx_v], tval_v, sem_in).wait()
        f = _TAIL_START + lanes
        g = f // _NN
        r = f - g * _NN
        i = r // _N
        j = r - i * _N
        idx = 16 + jnp.clip(g - ga, 0, 1) * 128 + (i // _NPC) * 16 + (j // _NPC)
        sc = plsc.load_gather(rp_v, [idx])
        tval_v[...] = tval_v[...] * sc
        pltpu.async_copy(tval_v, wout_hbm.at[tidx_v], sem_out).wait()


def _stage2_kernel(wp_ref, rn_ref, hp_ref, d_ref, theta_ref, hpk_ref):
    d_diag = jnp.sum(d_ref[...] * jnp.eye(_NPC, dtype=jnp.float32), axis=-1)
    r = lax.rsqrt(d_diag)
    a = (wp_ref[...] * rn_ref[...]
         * r[:, :, :, None] * r[:, :, None, :])
    a = a.reshape(_NG * _P, _NPC, _NPC)
    hp = hp_ref[...].reshape(_NG * _P, _NPC, _FDIM)
    ahp = lax.dot_general(a, hp,
                          (((2,), (1,)), ((0,), (0,))),
                          preferred_element_type=jnp.float32)
    hpk = lax.dot_general(ahp, theta_ref[...],
                          (((2,), (0,)), ((), ())),
                          preferred_element_type=jnp.float32)
    hpk_ref[...] = 0.1 * hpk.reshape(_NG, _P, _NPC, _FDIM)


def _stage2(Wp, Rn, Hp, D, theta):
    return pl.pallas_call(
        _stage2_kernel,
        out_shape=jax.ShapeDtypeStruct((_NG, _P, _NPC, _FDIM), jnp.float32),
    )(Wp, Rn, Hp, D, theta)


def kernel(Hc, Rc, Rcs, Wp, Rn, Hp, D, W, theta):
    s1 = jnp.concatenate(
        [Hc.reshape(-1), Rc.reshape(-1), Rcs.reshape(-1)])
    w_out_flat = _sc_rescale(s1, W.reshape(-1))
    hp_k = _stage2(Wp, Rn, Hp, D, theta)
    return (w_out_flat.reshape(_NG, _N, _N), hp_k)


# speedup vs baseline: 1.2683x; 1.0012x over previous
import functools

import jax
import jax.numpy as jnp
from jax import lax
from jax.experimental import pallas as pl
from jax.experimental.pallas import tpu as pltpu
from jax.experimental.pallas import tpu_sc as plsc

_NG, _P, _NPC, _FDIM = 10, 7, 23, 70
_N = _P * _NPC
_NN = _N * _N
_TOT = _NG * _NN
_GROUPS = 201
_GW = 8 * _N
_GBUF = _GW + 24
_GPT = 7
_LEFT_ROW = 8 * _GROUPS
_LEFT_OFF = _LEFT_ROW * _N
_TAIL_START = _TOT - 16
_NW = 32

_mesh = plsc.VectorSubcoreMesh(core_axis_name="c", subcore_axis_name="s")


@functools.partial(
    pl.kernel,
    mesh=_mesh,
    compiler_params=pltpu.CompilerParams(needs_layout_passes=False),
    out_type=jax.ShapeDtypeStruct((_TOT,), jnp.float32),
    scratch_types=[
        pltpu.VMEM((3 * _NG * 49,), jnp.float32),
        pltpu.VMEM((384,), jnp.float32),
        pltpu.VMEM((_GPT * _GBUF,), jnp.float32),
        pltpu.VMEM((128,), jnp.float32),
        pltpu.VMEM((128,), jnp.float32),
        pltpu.VMEM((16,), jnp.int32),
        pltpu.VMEM((16,), jnp.float32),
        pltpu.SemaphoreType.DMA,
        pltpu.SemaphoreType.DMA,
    ],
)
def _sc_rescale(s1_hbm, w_hbm, wout_hbm, s1_v,
                rp_v, wg_v, tmp_v, tmp2_v, tidx_v, tval_v, sem_in, sem_out):
    nc = 2
    wid = lax.axis_index("s") * nc + lax.axis_index("c")
    is_last = wid == _NW - 1
    sg = jnp.minimum(7 * wid, 6 * wid + 9)

    lanes = lax.broadcasted_iota(jnp.int32, (16,), 0)
    minl = jnp.minimum(lanes, 6)

    in_copies = []
    for gi in range(_GPT):
        gb = jnp.minimum(sg + gi, _GROUPS - 1)
        in_copies.append(pltpu.async_copy(
            w_hbm.at[pl.ds(gb * _GW, _GW)],
            wg_v.at[pl.ds(gi * _GBUF, _GW)], sem_in))

    pltpu.sync_copy(s1_hbm, s1_v)

    ga = jnp.clip((sg * 8) // _N, 0, _NG - 2)

    for m in range(2):
        base = (ga + m) * 49
        s = jnp.zeros((16,), jnp.float32)
        for r in range(7):
            s = s + plsc.load_gather(s1_v, [base + r * 7 + minl])
        tmp_v[pl.ds(16, 16)] = s
        dh = jnp.zeros((16,), jnp.float32)
        for k in range(7):
            sk = plsc.load_gather(tmp_v, [jnp.full((16,), 16 + k, jnp.int32)])
            dh = dh + sk * plsc.load_gather(s1_v, [490 + base + k * 7 + minl])
        dh = 0.1 * dh
        rdiag = plsc.load_gather(s1_v, [980 + base + minl * 8])
        t = dh / rdiag
        tmp2_v[pl.ds(16, 16)] = t
        for a in range(7):
            rc_row = plsc.load_gather(s1_v, [490 + base + a * 7 + minl])
            rc_col = plsc.load_gather(s1_v, [490 + base + minl * 7 + a])
            ta = plsc.load_gather(tmp2_v, [jnp.full((16,), 16 + a, jnp.int32)])
            row = 1.0 + t + (rc_col / rc_row) * ta
            row = jnp.where(lanes == a, 1.0, row)
            rp_v[pl.ds(16 + m * 128 + a * 16, 16)] = row

    def scale_vecs(rbase):
        out = []
        for v in range(10):
            cv = (16 * v) // _NPC
            cut = _NPC * (cv + 1) - 16 * v
            if cut >= 16:
                idxv = (rbase + cv) + lanes * 0
            else:
                idxv = rbase + jnp.where(lanes < cut, cv, cv + 1)
            out.append(plsc.load_gather(rp_v, [idxv]))
        out.append(plsc.load_gather(rp_v, [(rbase + 6) + lanes * 0]))
        return out

    def do_row(wofs, r):
        g = r // _N
        i = r - g * _N
        sv = scale_vecs(16 + jnp.clip(g - ga, 0, 1) * 128 + (i // _NPC) * 16)
        for v in range(10):
            w = wg_v[pl.ds(wofs + 16 * v, 16)]
            wg_v[pl.ds(wofs + 16 * v, 16)] = w * sv[v]
        w = wg_v[pl.ds(wofs + 160, 16)]
        plsc.store_scatter(wg_v, [wofs + 160 + lanes], w * sv[10],
                           mask=lanes < 1)

    out_copies = []
    for gi in range(_GPT):
        gb = jnp.minimum(sg + gi, _GROUPS - 1)
        r0 = gb * 8
        g_a = r0 // _N
        i_a = r0 - g_a * _N
        pi_a = i_a // _NPC
        rb_a = 16 + jnp.clip(g_a - ga, 0, 1) * 128 + pi_a * 16
        r7 = r0 + 7
        g_b = r7 // _N
        i_b = r7 - g_b * _N
        rb_b = 16 + jnp.clip(g_b - ga, 0, 1) * 128 + (i_b // _NPC) * 16
        kb = _NPC * (pi_a + 1) - i_a
        in_copies[gi].wait()
        sv_a = scale_vecs(rb_a)
        sv_b = scale_vecs(rb_b)
        for k in range(8):
            use_a = k < kb
            wofs = gi * _GBUF + k * _N
            for v in range(10):
                w = wg_v[pl.ds(wofs + 16 * v, 16)]
                wg_v[pl.ds(wofs + 16 * v, 16)] = (
                    w * jnp.where(use_a, sv_a[v], sv_b[v]))
            w = wg_v[pl.ds(wofs + 160, 16)]
            plsc.store_scatter(wg_v, [wofs + 160 + lanes],
                               w * jnp.where(use_a, sv_a[10], sv_b[10]),
                               mask=lanes < 1)
        out_copies.append(pltpu.async_copy(
            wg_v.at[pl.ds(gi * _GBUF, _GW)],
            wout_hbm.at[pl.ds(gb * _GW, _GW)], sem_out))
    for c in out_copies:
        c.wait()

    @pl.when(is_last)
    def _():
        pltpu.sync_copy(w_hbm.at[pl.ds(_LEFT_OFF, 320)],
                        wg_v.at[pl.ds(0, 320)])
        do_row(0, _LEFT_ROW)
        do_row(_N, _LEFT_ROW + 1)
        pltpu.sync_copy(wg_v.at[pl.ds(0, 320)],
                        wout_hbm.at[pl.ds(_LEFT_OFF, 320)])
        tidx_v[...] = _TAIL_START + lanes
        pltpu.async_copy(w_hbm.at[tidx_v], tval_v, sem_in).wait()
        f = _TAIL_START + lanes
        g = f // _NN
        r = f - g * _NN
        i = r // _N
        j = r - i * _N
        idx = 16 + jnp.clip(g - ga, 0, 1) * 128 + (i // _NPC) * 16 + (j // _NPC)
        sc = plsc.load_gather(rp_v, [idx])
        tval_v[...] = tval_v[...] * sc
        pltpu.async_copy(tval_v, wout_hbm.at[tidx_v], sem_out).wait()


def _stage2_kernel(wp_ref, rn_ref, hp_ref, d_ref, theta_ref, hpk_ref):
    d_diag = jnp.sum(d_ref[...] * jnp.eye(_NPC, dtype=jnp.float32), axis=-1)
    r = lax.rsqrt(d_diag)
    a = (wp_ref[...] * rn_ref[...]
         * r[:, :, :, None] * r[:, :, None, :])
    a = a.reshape(_NG * _P, _NPC, _NPC)
    hp = hp_ref[...].reshape(_NG * _P, _NPC, _FDIM)
    ahp = lax.dot_general(a, hp,
                          (((2,), (1,)), ((0,), (0,))),
                          preferred_element_type=jnp.float32)
    hpk = lax.dot_general(ahp, theta_ref[...],
                          (((2,), (0,)), ((), ())),
                          preferred_element_type=jnp.float32)
    hpk_ref[...] = 0.1 * hpk.reshape(_NG, _P, _NPC, _FDIM)


def _stage2(Wp, Rn, Hp, D, theta):
    return pl.pallas_call(
        _stage2_kernel,
        out_shape=jax.ShapeDtypeStruct((_NG, _P, _NPC, _FDIM), jnp.float32),
    )(Wp, Rn, Hp, D, theta)


def kernel(Hc, Rc, Rcs, Wp, Rn, Hp, D, W, theta):
    s1 = jnp.concatenate(
        [Hc.reshape(-1), Rc.reshape(-1), Rcs.reshape(-1)])
    w_out_flat = _sc_rescale(s1, W.reshape(-1))
    hp_k = _stage2(Wp, Rn, Hp, D, theta)
    return (w_out_flat.reshape(_NG, _N, _N), hp_k)
